# Initial kernel scaffold; baseline (speedup 1.0000x reference)
import jax
import jax.numpy as jnp
from jax.experimental import pallas as pl


def kernel(seg_feat, conv_weight, mask, ind, target, nums):
    raise NotImplementedError("write your pallas kernel here")


# speedup vs baseline: 628.7435x; 628.7435x over previous
import functools

import jax
import jax.numpy as jnp
from jax import lax
from jax.experimental import pallas as pl
from jax.experimental.pallas import tpu as pltpu
from jax.experimental.pallas import tpu_sc as plsc

_NW = 32
_EPW = 1408
_CHUNK = 128
_NCH = _EPW // _CHUNK


def _sc_gather_body(table_hbm, addr_hbm, out_hbm, addr_v, outv, sem):
    wid = lax.axis_index("s") * 2 + lax.axis_index("c")
    base = wid * _EPW
    pltpu.sync_copy(addr_hbm.at[pl.ds(base, _EPW)], addr_v)
    cps = [
        pltpu.async_copy(
            table_hbm.at[addr_v.at[pl.ds(ck * _CHUNK, _CHUNK)]],
            outv.at[pl.ds(ck * _CHUNK, _CHUNK)],
            sem,
        )
        for ck in range(_NCH)
    ]
    for cp in cps:
        cp.wait()
    pltpu.sync_copy(outv, out_hbm.at[pl.ds(base, _EPW)])


def _main_body(gsrc_ref, gbsel_ref, ngv_ref, f_ref, t_ref, w0_ref, w1_ref,
               w2_ref, out_ref, acc_ref):
    k = pl.program_id(0)

    @pl.when(k == 0)
    def _init():
        acc_ref[...] = jnp.zeros((24, 16384), jnp.float32)

    @pl.when(k < ngv_ref[0])
    def _compute():
        fa = f_ref[0]
        h0 = jnp.maximum(
            jnp.dot(w0_ref[0], fa, preferred_element_type=jnp.float32), 0.0)
        h1 = jnp.maximum(
            jnp.dot(w1_ref[0], h0, preferred_element_type=jnp.float32), 0.0)
        w2g = w2_ref[0]
        logit = jnp.dot(w2g, h1, preferred_element_type=jnp.float32)
        s = 1.0 / (1.0 + jnp.exp(-logit))
        t = t_ref[0]
        mask = (w2g[:, 64:65] > -1e8).astype(jnp.float32)
        acc_ref[0:8, :] += s * t
        acc_ref[8:16, :] += s * s
        acc_ref[16:24, :] += (t * mask) * t

    @pl.when(k == pl.num_programs(0) - 1)
    def _finish():
        inter = jnp.sum(acc_ref[0:8, :])
        ss = jnp.sum(acc_ref[8:16, :])
        tt = jnp.sum(acc_ref[16:24, :])
        loss = (1.0 - (2.0 * inter + 1.0) / (ss + tt + 1.0)) * 0.25
        out_ref[...] = jnp.full((1, 128), loss, jnp.float32)


def kernel(seg_feat, conv_weight, mask, ind, target, nums):
    n, c, h, w = seg_feat.shape
    m = target.shape[1]
    hw = h * w
    nm = n * m
    G = 8
    ng = nm // G
    gpb = m // G

    ind64 = ind[:, :m].astype(jnp.int32)
    indf = ind64.reshape(-1)

    binst = jnp.arange(nm, dtype=jnp.int32) // m
    chpad = jnp.minimum(jnp.arange(176, dtype=jnp.int32), 168)
    addr = (binst[:, None] * 169 + chpad[None, :]) * hw + indf[:, None]
    addrf = addr.reshape(-1).astype(jnp.int32)
    table = conv_weight.reshape(-1)

    mesh = plsc.VectorSubcoreMesh(core_axis_name="c", subcore_axis_name="s")
    sc_gather = pl.kernel(
        _sc_gather_body,
        mesh=mesh,
        out_type=jax.ShapeDtypeStruct((nm * 176,), jnp.float32),
        scratch_types=[
            pltpu.VMEM((_EPW,), jnp.int32),
            pltpu.VMEM((_EPW,), jnp.float32),
            pltpu.SemaphoreType.DMA,
        ],
    )
    wg = sc_gather(table, addrf).reshape(nm, 176)

    w0blk = wg[:, :80].reshape(nm, 8, 10)
    w1blk = wg[:, 80:144].reshape(nm, 8, 8)
    w2blk = wg[:, 144:152]
    b0 = wg[:, 152:160]
    b1 = wg[:, 160:168]
    b2 = wg[:, 168]
    indf = ind64.reshape(-1)
    xi = (indf % w).astype(jnp.float32)
    yi = indf.astype(jnp.float32) / jnp.float32(w)
    b0p = b0 - (w0blk[:, :, 8] * xi[:, None]
                + w0blk[:, :, 9] * yi[:, None]) * (1.0 / w)

    f32 = jnp.float32
    gd = 8 * G
    w0rows = jnp.concatenate([w0blk, b0p[:, :, None]], axis=2)
    onesrow0 = jnp.broadcast_to(
        (jnp.arange(11) == 10).astype(f32)[None, None, :], (ng, 1, 11))
    w0aug = jnp.concatenate([w0rows.reshape(ng, gd, 11), onesrow0], axis=1)

    E = jnp.eye(gd, dtype=f32).reshape(G, 8, gd)
    w1r = w1blk.reshape(ng, G, 8, 8)
    w1bd = jnp.einsum('gsrc,srp,scq->gpq', w1r, E, E)
    b1col = b1.reshape(ng, gd, 1)
    lastrow = jnp.broadcast_to(
        (jnp.arange(gd + 1) == gd).astype(f32)[None, None, :], (ng, 1, gd + 1))
    w1aug = jnp.concatenate(
        [jnp.concatenate([w1bd, b1col], axis=2), lastrow], axis=1)

    valid = (jnp.arange(m)[None, :] < nums[:, None]).reshape(-1)
    b2m = jnp.where(valid, b2, f32(-1e9))
    w2bd = jnp.einsum('gsc,scq->gsq', w2blk.reshape(ng, G, 8), E)
    w2aug = jnp.concatenate([w2bd, b2m.reshape(ng, G, 1)], axis=2)

    jf = jnp.arange(hw, dtype=jnp.int32)
    xpl = (jf % w).astype(f32) * (1.0 / w)
    ypl = (jf // w).astype(f32) * (1.0 / w)
    segaug = jnp.concatenate([
        seg_feat.reshape(n, c, hw),
        jnp.broadcast_to(xpl[None, None, :], (n, 1, hw)),
        jnp.broadcast_to(ypl[None, None, :], (n, 1, hw)),
        jnp.ones((n, 1, hw), f32),
    ], axis=1)

    gval = (jnp.arange(gpb)[None, :] * G < nums[:, None]).reshape(-1)
    order_g = jnp.argsort(jnp.where(gval, 0, 1).astype(jnp.int32),
                          stable=True).astype(jnp.int32)
    ngv = jnp.sum(gval).astype(jnp.int32)
    ks = jnp.arange(ng, dtype=jnp.int32)
    gsrc = jnp.where(ks < ngv, order_g, 0).astype(jnp.int32)
    gbsel = jnp.where(ks < ngv, order_g // gpb, n - 1).astype(jnp.int32)

    tgt = target.reshape(ng, G, hw)

    grid_spec = pltpu.PrefetchScalarGridSpec(
        num_scalar_prefetch=3,
        grid=(ng,),
        in_specs=[
            pl.BlockSpec((1, 11, hw), lambda p, gs, gb, v: (gb[p], 0, 0)),
            pl.BlockSpec((1, G, hw), lambda p, gs, gb, v: (gs[p], 0, 0)),
            pl.BlockSpec((1, 8 * G + 1, 11), lambda p, gs, gb, v: (gs[p], 0, 0)),
            pl.BlockSpec((1, 8 * G + 1, 8 * G + 1),
                         lambda p, gs, gb, v: (gs[p], 0, 0)),
            pl.BlockSpec((1, G, 8 * G + 1), lambda p, gs, gb, v: (gs[p], 0, 0)),
        ],
        out_specs=pl.BlockSpec((1, 128), lambda p, gs, gb, v: (0, 0)),
        scratch_shapes=[pltpu.VMEM((24, hw), jnp.float32)],
    )
    loss2d = pl.pallas_call(
        _main_body,
        grid_spec=grid_spec,
        out_shape=jax.ShapeDtypeStruct((1, 128), jnp.float32),
        interpret=False,
    )(gsrc, gbsel, ngv.reshape(1), segaug, tgt, w0aug, w1aug, w2aug)
    return loss2d[0, 0]
